# Initial kernel scaffold; baseline (speedup 1.0000x reference)
import jax
import jax.numpy as jnp
from jax.experimental import pallas as pl


def kernel(x, token_table, pos_table):
    raise NotImplementedError("write your pallas kernel here")


# speedup vs baseline: 1.3655x; 1.3655x over previous
import functools

import jax
import jax.numpy as jnp
from jax import lax
from jax.experimental import pallas as pl
from jax.experimental.pallas import tpu as pltpu
from jax.experimental.pallas import tpu_sc as plsc

EMBED = 32
SEQ = 200
LANES = 16
IDX_MINOR = 100
ROWS_PER_CHUNK = 800


def _embed_kernel(rows_per_worker, x_hbm, tok_hbm, pos_hbm, out_hbm,
                  idx_v, rows_v, pos_v, gat_sem):
    wid = lax.axis_index("s") * 2 + lax.axis_index("c")
    base_row = wid * rows_per_worker
    n_chunks = rows_per_worker // ROWS_PER_CHUNK
    idx_rows_per_chunk = ROWS_PER_CHUNK // IDX_MINOR

    pltpu.sync_copy(pos_hbm, pos_v)

    def chunk_body(c, carry):
        row0 = pl.multiple_of(base_row + c * ROWS_PER_CHUNK, ROWS_PER_CHUNK)
        idx_row0 = pl.multiple_of(row0 // IDX_MINOR, 8)
        pltpu.sync_copy(x_hbm.at[pl.ds(idx_row0, idx_rows_per_chunk)], idx_v)
        cps = []
        for g in range(idx_rows_per_chunk):
            cps.append(pltpu.make_async_copy(
                tok_hbm.at[idx_v.at[g]],
                rows_v.at[pl.ds(g * IDX_MINOR, IDX_MINOR)], gat_sem))
            cps[-1].start()
        for cp in cps:
            cp.wait()

        def add_body(i, carry2):
            p0 = pos_v[i, pl.ds(0, LANES)]
            p1 = pos_v[i, pl.ds(LANES, LANES)]
            for s in range(ROWS_PER_CHUNK // SEQ):
                r = s * SEQ + i
                rows_v[r, pl.ds(0, LANES)] = rows_v[r, pl.ds(0, LANES)] + p0
                rows_v[r, pl.ds(LANES, LANES)] = (
                    rows_v[r, pl.ds(LANES, LANES)] + p1)
            return carry2

        lax.fori_loop(0, SEQ, add_body, 0, unroll=2)

        pltpu.sync_copy(rows_v, out_hbm.at[pl.ds(row0, ROWS_PER_CHUNK)])
        return carry

    lax.fori_loop(0, n_chunks, chunk_body, 0)


def kernel(x, token_table, pos_table):
    batch, seq = x.shape
    n_rows = batch * seq
    num_workers = 32
    rows_per_worker = n_rows // num_workers
    x_flat = x.reshape(n_rows // IDX_MINOR, IDX_MINOR).astype(jnp.int32)

    mesh = plsc.VectorSubcoreMesh(core_axis_name="c", subcore_axis_name="s")
    run = pl.kernel(
        functools.partial(_embed_kernel, rows_per_worker),
        mesh=mesh,
        out_type=jax.ShapeDtypeStruct((n_rows, EMBED), jnp.float32),
        scratch_types=[
            pltpu.VMEM((ROWS_PER_CHUNK // IDX_MINOR, IDX_MINOR), jnp.int32),
            pltpu.VMEM((ROWS_PER_CHUNK, EMBED), jnp.float32),
            pltpu.VMEM((SEQ, EMBED), jnp.float32),
            pltpu.SemaphoreType.DMA,
        ],
        compiler_params=pltpu.CompilerParams(use_tc_tiling_on_sc=False),
    )
    out = run(x_flat, token_table, pos_table)
    return out.reshape(batch, seq, EMBED)
